# baseline (speedup 1.0000x reference)
import jax
import jax.numpy as jnp
from jax import lax
from jax.experimental import pallas as pl
from jax.experimental.pallas import tpu as pltpu
from jax.experimental.pallas import tpu_sc as plsc

_INT_MIN = -2147483648
_INT_MAX = 2147483647
_L = 16
_T = 8192
_CHUNK = 1024


def _scores_kernel(x_ref, w1_ref, b1_ref, w2_ref, out_ref):
    x = x_ref[...].astype(jnp.bfloat16)
    h = jnp.tanh(
        jax.lax.dot_general(x, w1_ref[...].astype(jnp.bfloat16),
                            (((1,), (1,)), ((), ())),
                            preferred_element_type=jnp.float32)
        + b1_ref[...])
    hb = h.astype(jnp.bfloat16).astype(jnp.float32)
    v2 = w2_ref[...].astype(jnp.bfloat16).astype(jnp.float32)
    s = jnp.sum(hb * v2, axis=1)
    out_ref[...] = s[:, None]


def _thresh_kernel(s_ref, ti_ref, tf_ref):
    s = s_ref[...]
    B = s.shape[0]
    T = s.shape[1] * s.shape[2]
    n_mask = T // 2
    bits = jax.lax.bitcast_convert_type(s, jnp.int32)
    key = jnp.where(bits >= 0, bits, jnp.int32(_INT_MIN) - bits)

    def vbody(_, lohi):
        lo, hi = lohi
        mid = (lo >> 1) + (hi >> 1) + (lo & hi & 1)
        c = jnp.sum((key <= mid).astype(jnp.int32), axis=(1, 2),
                    keepdims=True)
        ge = c >= n_mask
        return (jnp.where(ge, lo, mid + 1), jnp.where(ge, mid, hi))

    lo0 = jnp.full((B, 1, 1), _INT_MIN, jnp.int32)
    hi0 = jnp.full((B, 1, 1), _INT_MAX, jnp.int32)
    K, _ = jax.lax.fori_loop(0, 32, vbody, (lo0, hi0))

    cnt_lt = jnp.sum((key < K).astype(jnp.int32), axis=(1, 2), keepdims=True)
    n_eq = n_mask - cnt_lt
    i1 = jax.lax.broadcasted_iota(jnp.int32, s.shape, 1)
    i2 = jax.lax.broadcasted_iota(jnp.int32, s.shape, 2)
    gidx = i1 * s.shape[2] + i2
    eq = key == K

    def ibody(_, lohi):
        lo, hi = lohi
        mid = (lo + hi) >> 1
        c = jnp.sum((eq & (gidx <= mid)).astype(jnp.int32), axis=(1, 2),
                    keepdims=True)
        ge = c >= n_eq
        return (jnp.where(ge, lo, mid + 1), jnp.where(ge, mid, hi))

    lo0i = jnp.zeros((B, 1, 1), jnp.int32)
    hi0i = jnp.full((B, 1, 1), T - 1, jnp.int32)
    I, _ = jax.lax.fori_loop(0, 13, ibody, (lo0i, hi0i))

    M = jnp.max(s, axis=(1, 2), keepdims=True)
    lane = jax.lax.broadcasted_iota(jnp.int32, (B, _L), 1)
    Kb = jnp.broadcast_to(K[:, :, 0], (B, _L))
    Ib = jnp.broadcast_to(I[:, :, 0], (B, _L))
    Mb = jnp.broadcast_to(M[:, :, 0], (B, _L))
    thr_bits = jnp.where(Kb >= 0, Kb, jnp.int32(_INT_MIN) - Kb)
    thr_f = jax.lax.bitcast_convert_type(thr_bits, jnp.float32)
    ti_ref[...] = jnp.where(lane == 0, Kb, jnp.where(lane == 1, Ib, 0))
    tf_ref[...] = jnp.where(lane == 0, Mb, jnp.where(lane == 1, thr_f, 0.0))


def _sc_weights_body(scores_hbm, ti_hbm, tf_hbm, out_hbm, sv, ev,
                     stg_i, stg_f, stg8f, sp_sum):
    c = lax.axis_index("c")
    s_id = lax.axis_index("s")
    rc = s_id // 8
    j = s_id % 8
    r = 2 * c + rc
    chunk_base = pl.multiple_of(r * _T + j * _CHUNK, _CHUNK)
    n_vec = _CHUNK // _L

    pltpu.sync_copy(scores_hbm.at[pl.ds(chunk_base, _CHUNK)], sv)
    pltpu.sync_copy(ti_hbm.at[pl.ds(pl.multiple_of(r * _L, _L), _L)], stg_i)
    pltpu.sync_copy(tf_hbm.at[pl.ds(pl.multiple_of(r * _L, _L), _L)], stg_f)
    tvec_i = stg_i[pl.ds(0, _L)]
    I = tvec_i[1]
    tvec_f = stg_f[pl.ds(0, _L)]
    M = tvec_f[0]
    thrf = tvec_f[1]

    def _lane_reduce(vec, op):
        acc = vec[0]
        for ln in range(1, _L):
            acc = op(acc, vec[ln])
        return acc

    def body_exp(i, acc):
        v = sv[pl.ds(i * _L, _L)]
        gidx = lax.iota(jnp.int32, _L) + (j * _CHUNK + i * _L)
        masked = (v < thrf) | ((v == thrf) & (gidx <= I))
        e = jnp.where(masked, 0.0, jnp.exp(v - M))
        ev[pl.ds(i * _L, _L)] = e
        return acc + e

    sumv = lax.fori_loop(0, n_vec, body_exp, jnp.zeros((_L,), jnp.float32))
    stg_f[pl.ds(0, _L)] = jnp.full((_L,), _lane_reduce(sumv, jnp.add),
                                   jnp.float32)
    pltpu.sync_copy(stg_f, sp_sum.at[pl.ds(pl.multiple_of((rc * 8 + j) * _L, _L), _L)])
    plsc.subcore_barrier()

    pltpu.sync_copy(sp_sum.at[pl.ds(pl.multiple_of(rc * 8 * _L, _L), 8 * _L)], stg8f)
    S = stg8f[pl.ds(0, _L)][0]
    for jj in range(1, 8):
        S = S + stg8f[pl.ds(jj * _L, _L)][0]
    svec = jnp.full((_L,), S, jnp.float32)

    def body_norm(i, carry):
        ev[pl.ds(i * _L, _L)] = ev[pl.ds(i * _L, _L)] / svec
        return carry

    lax.fori_loop(0, n_vec, body_norm, jnp.int32(0))
    pltpu.sync_copy(ev, out_hbm.at[pl.ds(chunk_base, _CHUNK)])


def _sc_weights(scores_flat, ti_flat, tf_flat):
    return pl.kernel(
        _sc_weights_body,
        out_type=jax.ShapeDtypeStruct((4 * _T,), jnp.float32),
        mesh=plsc.VectorSubcoreMesh(core_axis_name="c", subcore_axis_name="s"),
        scratch_types=[
            pltpu.VMEM((_CHUNK,), jnp.float32),
            pltpu.VMEM((_CHUNK,), jnp.float32),
            pltpu.VMEM((_L,), jnp.int32),
            pltpu.VMEM((_L,), jnp.float32),
            pltpu.VMEM((8 * _L,), jnp.float32),
            pltpu.VMEM_SHARED((2 * 8 * _L,), jnp.float32),
        ],
    )(scores_flat, ti_flat, tf_flat)


def _scale_kernel(x_ref, w_ref, o_ref):
    o_ref[...] = x_ref[...] * w_ref[...]


def kernel(x, W1, b1, w2, b2):
    B, T, D = x.shape
    del b2
    xf = x.reshape(B * T, D)

    BT = 512
    scores = pl.pallas_call(
        _scores_kernel,
        grid=(B * T // BT,),
        in_specs=[
            pl.BlockSpec((BT, D), lambda i: (i, 0)),
            pl.BlockSpec((D, D), lambda i: (0, 0)),
            pl.BlockSpec((1, D), lambda i: (0, 0)),
            pl.BlockSpec((1, D), lambda i: (0, 0)),
        ],
        out_specs=pl.BlockSpec((BT, 1), lambda i: (i, 0)),
        out_shape=jax.ShapeDtypeStruct((B * T, 1), jnp.float32),
    )(xf, W1, b1.reshape(1, D), w2.reshape(1, D))

    ti, tf = pl.pallas_call(
        _thresh_kernel,
        out_shape=(jax.ShapeDtypeStruct((B, _L), jnp.int32),
                   jax.ShapeDtypeStruct((B, _L), jnp.float32)),
    )(scores.reshape(B, T // 128, 128))
    weights = _sc_weights(scores.reshape(B * T), ti.reshape(B * _L),
                          tf.reshape(B * _L)).reshape(B, T)

    R = 1024
    out = pl.pallas_call(
        _scale_kernel,
        grid=(B * T // R,),
        in_specs=[
            pl.BlockSpec((R, D), lambda i: (i, 0)),
            pl.BlockSpec((R, 1), lambda i: (i, 0)),
        ],
        out_specs=pl.BlockSpec((R, D), lambda i: (i, 0)),
        out_shape=jax.ShapeDtypeStruct((B * T, D), jnp.float32),
    )(xf, weights.reshape(B * T, 1))

    return out.reshape(B, T, D), weights


# speedup vs baseline: 1.1205x; 1.1205x over previous
import jax
import jax.numpy as jnp
from jax import lax
from jax.experimental import pallas as pl
from jax.experimental.pallas import tpu as pltpu
from jax.experimental.pallas import tpu_sc as plsc

_INT_MIN = -2147483648
_INT_MAX = 2147483647
_L = 16
_T = 8192
_CHUNK = 1024


def _scores_kernel(x_ref, w1_ref, b1_ref, w2_ref, out_ref):
    x = x_ref[...].astype(jnp.bfloat16)
    h = jnp.tanh(
        jax.lax.dot_general(x, w1_ref[...].astype(jnp.bfloat16),
                            (((1,), (1,)), ((), ())),
                            preferred_element_type=jnp.float32)
        + b1_ref[...])
    hb = h.astype(jnp.bfloat16).astype(jnp.float32)
    v2 = w2_ref[...].astype(jnp.bfloat16).astype(jnp.float32)
    s = jnp.sum(hb * v2, axis=1)
    out_ref[...] = s[:, None]


def _thresh_kernel(s_ref, ti_ref, tf_ref):
    s = s_ref[...]
    B = s.shape[0]
    T = s.shape[1] * s.shape[2]
    n_mask = T // 2
    bits = jax.lax.bitcast_convert_type(s, jnp.int32)
    key = jnp.where(bits >= 0, bits, jnp.int32(_INT_MIN) - bits)

    def vbody(_, lohi):
        lo, hi = lohi
        mid = (lo >> 1) + (hi >> 1) + (lo & hi & 1)
        c = jnp.sum((key <= mid).astype(jnp.int32), axis=(1, 2),
                    keepdims=True)
        ge = c >= n_mask
        return (jnp.where(ge, lo, mid + 1), jnp.where(ge, mid, hi))

    lo0 = jnp.full((B, 1, 1), _INT_MIN, jnp.int32)
    hi0 = jnp.full((B, 1, 1), _INT_MAX, jnp.int32)
    K, _ = jax.lax.fori_loop(0, 32, vbody, (lo0, hi0))

    cnt_lt = jnp.sum((key < K).astype(jnp.int32), axis=(1, 2), keepdims=True)
    n_eq = n_mask - cnt_lt
    i1 = jax.lax.broadcasted_iota(jnp.int32, s.shape, 1)
    i2 = jax.lax.broadcasted_iota(jnp.int32, s.shape, 2)
    gidx = i1 * s.shape[2] + i2
    eq = key == K

    def ibody(_, lohi):
        lo, hi = lohi
        mid = (lo + hi) >> 1
        c = jnp.sum((eq & (gidx <= mid)).astype(jnp.int32), axis=(1, 2),
                    keepdims=True)
        ge = c >= n_eq
        return (jnp.where(ge, lo, mid + 1), jnp.where(ge, mid, hi))

    lo0i = jnp.zeros((B, 1, 1), jnp.int32)
    hi0i = jnp.full((B, 1, 1), T - 1, jnp.int32)
    I, _ = jax.lax.fori_loop(0, 13, ibody, (lo0i, hi0i))

    M = jnp.max(s, axis=(1, 2), keepdims=True)
    lane = jax.lax.broadcasted_iota(jnp.int32, (B, _L), 1)
    Kb = jnp.broadcast_to(K[:, :, 0], (B, _L))
    Ib = jnp.broadcast_to(I[:, :, 0], (B, _L))
    Mb = jnp.broadcast_to(M[:, :, 0], (B, _L))
    thr_bits = jnp.where(Kb >= 0, Kb, jnp.int32(_INT_MIN) - Kb)
    thr_f = jax.lax.bitcast_convert_type(thr_bits, jnp.float32)
    ti_ref[...] = jnp.where(lane == 0, Kb, jnp.where(lane == 1, Ib, 0))
    tf_ref[...] = jnp.where(lane == 0, Mb, jnp.where(lane == 1, thr_f, 0.0))


def _sc_weights_body(scores_hbm, ti_hbm, tf_hbm, out_hbm, sv, ev,
                     stg_i, stg_f, stg8f, sp_sum):
    c = lax.axis_index("c")
    s_id = lax.axis_index("s")
    rc = s_id // 8
    j = s_id % 8
    r = 2 * c + rc
    chunk_base = pl.multiple_of(r * _T + j * _CHUNK, _CHUNK)
    n_vec = _CHUNK // _L

    pltpu.sync_copy(scores_hbm.at[pl.ds(chunk_base, _CHUNK)], sv)
    pltpu.sync_copy(ti_hbm.at[pl.ds(pl.multiple_of(r * _L, _L), _L)], stg_i)
    pltpu.sync_copy(tf_hbm.at[pl.ds(pl.multiple_of(r * _L, _L), _L)], stg_f)
    tvec_i = stg_i[pl.ds(0, _L)]
    I = tvec_i[1]
    tvec_f = stg_f[pl.ds(0, _L)]
    M = tvec_f[0]
    thrf = tvec_f[1]

    def _lane_reduce(vec, op):
        acc = vec[0]
        for ln in range(1, _L):
            acc = op(acc, vec[ln])
        return acc

    def body_exp(i, acc):
        v = sv[pl.ds(i * _L, _L)]
        gidx = lax.iota(jnp.int32, _L) + (j * _CHUNK + i * _L)
        masked = (v < thrf) | ((v == thrf) & (gidx <= I))
        e = jnp.where(masked, 0.0, jnp.exp(v - M))
        ev[pl.ds(i * _L, _L)] = e
        return acc + e

    sumv = lax.fori_loop(0, n_vec, body_exp, jnp.zeros((_L,), jnp.float32))
    stg_f[pl.ds(0, _L)] = jnp.full((_L,), _lane_reduce(sumv, jnp.add),
                                   jnp.float32)
    pltpu.sync_copy(stg_f, sp_sum.at[pl.ds(pl.multiple_of((rc * 8 + j) * _L, _L), _L)])
    plsc.subcore_barrier()

    pltpu.sync_copy(sp_sum.at[pl.ds(pl.multiple_of(rc * 8 * _L, _L), 8 * _L)], stg8f)
    S = stg8f[pl.ds(0, _L)][0]
    for jj in range(1, 8):
        S = S + stg8f[pl.ds(jj * _L, _L)][0]
    svec = jnp.full((_L,), S, jnp.float32)

    def body_norm(i, carry):
        ev[pl.ds(i * _L, _L)] = ev[pl.ds(i * _L, _L)] / svec
        return carry

    lax.fori_loop(0, n_vec, body_norm, jnp.int32(0))
    pltpu.sync_copy(ev, out_hbm.at[pl.ds(chunk_base, _CHUNK)])


def _sc_weights(scores_flat, ti_flat, tf_flat):
    return pl.kernel(
        _sc_weights_body,
        out_type=jax.ShapeDtypeStruct((4 * _T,), jnp.float32),
        mesh=plsc.VectorSubcoreMesh(core_axis_name="c", subcore_axis_name="s"),
        scratch_types=[
            pltpu.VMEM((_CHUNK,), jnp.float32),
            pltpu.VMEM((_CHUNK,), jnp.float32),
            pltpu.VMEM((_L,), jnp.int32),
            pltpu.VMEM((_L,), jnp.float32),
            pltpu.VMEM((8 * _L,), jnp.float32),
            pltpu.VMEM_SHARED((2 * 8 * _L,), jnp.float32),
        ],
    )(scores_flat, ti_flat, tf_flat)


def _scale_kernel(x_ref, w_ref, o_ref):
    o_ref[...] = x_ref[...] * w_ref[...]


def kernel(x, W1, b1, w2, b2):
    B, T, D = x.shape
    del b2
    xf = x.reshape(B * T, D)

    BT = 1024
    scores = pl.pallas_call(
        _scores_kernel,
        grid=(B * T // BT,),
        in_specs=[
            pl.BlockSpec((BT, D), lambda i: (i, 0)),
            pl.BlockSpec((D, D), lambda i: (0, 0)),
            pl.BlockSpec((1, D), lambda i: (0, 0)),
            pl.BlockSpec((1, D), lambda i: (0, 0)),
        ],
        out_specs=pl.BlockSpec((BT, 1), lambda i: (i, 0)),
        out_shape=jax.ShapeDtypeStruct((B * T, 1), jnp.float32),
    )(xf, W1, b1.reshape(1, D), w2.reshape(1, D))

    ti, tf = pl.pallas_call(
        _thresh_kernel,
        out_shape=(jax.ShapeDtypeStruct((B, _L), jnp.int32),
                   jax.ShapeDtypeStruct((B, _L), jnp.float32)),
    )(scores.reshape(B, T // 128, 128))
    weights = _sc_weights(scores.reshape(B * T), ti.reshape(B * _L),
                          tf.reshape(B * _L)).reshape(B, T)

    R = 1024
    out = pl.pallas_call(
        _scale_kernel,
        grid=(B * T // R,),
        in_specs=[
            pl.BlockSpec((R, D), lambda i: (i, 0)),
            pl.BlockSpec((R, 1), lambda i: (i, 0)),
        ],
        out_specs=pl.BlockSpec((R, D), lambda i: (i, 0)),
        out_shape=jax.ShapeDtypeStruct((B * T, D), jnp.float32),
    )(xf, weights.reshape(B * T, 1))

    return out.reshape(B, T, D), weights


# speedup vs baseline: 1.1713x; 1.0453x over previous
import jax
import jax.numpy as jnp
from jax import lax
from jax.experimental import pallas as pl
from jax.experimental.pallas import tpu as pltpu
from jax.experimental.pallas import tpu_sc as plsc

_INT_MIN = -2147483648
_INT_MAX = 2147483647
_L = 16
_T = 8192
_CHUNK = 1024


def _scores_kernel(x_ref, w1_ref, b1_ref, w2_ref, out_ref):
    x = x_ref[...].astype(jnp.bfloat16)
    h = jnp.tanh(
        jax.lax.dot_general(x, w1_ref[...].astype(jnp.bfloat16),
                            (((1,), (1,)), ((), ())),
                            preferred_element_type=jnp.float32)
        + b1_ref[...])
    hb = h.astype(jnp.bfloat16).astype(jnp.float32)
    v2 = w2_ref[...].astype(jnp.bfloat16).astype(jnp.float32)
    s = jnp.sum(hb * v2, axis=1)
    out_ref[...] = s[:, None]


def _thresh_kernel(s_ref, ti_ref, tf_ref):
    s = s_ref[...]
    B = s.shape[0]
    T = s.shape[1] * s.shape[2]
    n_mask = T // 2
    bits = jax.lax.bitcast_convert_type(s, jnp.int32)
    key = jnp.where(bits >= 0, bits, jnp.int32(_INT_MIN) - bits)

    def vbody(_, lohi):
        lo, hi = lohi
        mid = (lo >> 1) + (hi >> 1) + (lo & hi & 1)
        c = jnp.sum((key <= mid).astype(jnp.int32), axis=(1, 2),
                    keepdims=True)
        ge = c >= n_mask
        return (jnp.where(ge, lo, mid + 1), jnp.where(ge, mid, hi))

    lo0 = jnp.full((B, 1, 1), _INT_MIN, jnp.int32)
    hi0 = jnp.full((B, 1, 1), _INT_MAX, jnp.int32)
    K, _ = jax.lax.fori_loop(0, 32, vbody, (lo0, hi0))

    cnt_lt = jnp.sum((key < K).astype(jnp.int32), axis=(1, 2), keepdims=True)
    n_eq = n_mask - cnt_lt
    i1 = jax.lax.broadcasted_iota(jnp.int32, s.shape, 1)
    i2 = jax.lax.broadcasted_iota(jnp.int32, s.shape, 2)
    gidx = i1 * s.shape[2] + i2
    eq = key == K

    def ibody(_, lohi):
        lo, hi = lohi
        mid = (lo + hi) >> 1
        c = jnp.sum((eq & (gidx <= mid)).astype(jnp.int32), axis=(1, 2),
                    keepdims=True)
        ge = c >= n_eq
        return (jnp.where(ge, lo, mid + 1), jnp.where(ge, mid, hi))

    lo0i = jnp.zeros((B, 1, 1), jnp.int32)
    hi0i = jnp.full((B, 1, 1), T - 1, jnp.int32)
    I, _ = jax.lax.fori_loop(0, 13, ibody, (lo0i, hi0i))

    M = jnp.max(s, axis=(1, 2), keepdims=True)
    lane = jax.lax.broadcasted_iota(jnp.int32, (B, _L), 1)
    Kb = jnp.broadcast_to(K[:, :, 0], (B, _L))
    Ib = jnp.broadcast_to(I[:, :, 0], (B, _L))
    Mb = jnp.broadcast_to(M[:, :, 0], (B, _L))
    thr_bits = jnp.where(Kb >= 0, Kb, jnp.int32(_INT_MIN) - Kb)
    thr_f = jax.lax.bitcast_convert_type(thr_bits, jnp.float32)
    ti_ref[...] = jnp.where(lane == 0, Kb, jnp.where(lane == 1, Ib, 0))
    tf_ref[...] = jnp.where(lane == 0, Mb, jnp.where(lane == 1, thr_f, 0.0))


def _sc_weights_body(scores_hbm, ti_hbm, tf_hbm, out_hbm, sv, ev,
                     stg_i, stg_f, stg8f, sp_sum):
    c = lax.axis_index("c")
    s_id = lax.axis_index("s")
    rc = s_id // 8
    j = s_id % 8
    r = 2 * c + rc
    chunk_base = pl.multiple_of(r * _T + j * _CHUNK, _CHUNK)
    n_vec = _CHUNK // _L

    pltpu.sync_copy(scores_hbm.at[pl.ds(chunk_base, _CHUNK)], sv)
    pltpu.sync_copy(ti_hbm.at[pl.ds(pl.multiple_of(r * _L, _L), _L)], stg_i)
    pltpu.sync_copy(tf_hbm.at[pl.ds(pl.multiple_of(r * _L, _L), _L)], stg_f)
    tvec_i = stg_i[pl.ds(0, _L)]
    I = tvec_i[1]
    tvec_f = stg_f[pl.ds(0, _L)]
    M = tvec_f[0]
    thrf = tvec_f[1]

    def _lane_reduce(vec, op):
        acc = vec[0]
        for ln in range(1, _L):
            acc = op(acc, vec[ln])
        return acc

    def body_exp(i, acc):
        v = sv[pl.ds(i * _L, _L)]
        gidx = lax.iota(jnp.int32, _L) + (j * _CHUNK + i * _L)
        masked = (v < thrf) | ((v == thrf) & (gidx <= I))
        e = jnp.where(masked, 0.0, jnp.exp(v - M))
        ev[pl.ds(i * _L, _L)] = e
        return acc + e

    sumv = lax.fori_loop(0, n_vec, body_exp, jnp.zeros((_L,), jnp.float32))
    stg_f[pl.ds(0, _L)] = jnp.full((_L,), _lane_reduce(sumv, jnp.add),
                                   jnp.float32)
    pltpu.sync_copy(stg_f, sp_sum.at[pl.ds(pl.multiple_of((rc * 8 + j) * _L, _L), _L)])
    plsc.subcore_barrier()

    pltpu.sync_copy(sp_sum.at[pl.ds(pl.multiple_of(rc * 8 * _L, _L), 8 * _L)], stg8f)
    S = stg8f[pl.ds(0, _L)][0]
    for jj in range(1, 8):
        S = S + stg8f[pl.ds(jj * _L, _L)][0]
    svec = jnp.full((_L,), S, jnp.float32)

    def body_norm(i, carry):
        ev[pl.ds(i * _L, _L)] = ev[pl.ds(i * _L, _L)] / svec
        return carry

    lax.fori_loop(0, n_vec, body_norm, jnp.int32(0))
    pltpu.sync_copy(ev, out_hbm.at[pl.ds(chunk_base, _CHUNK)])


def _sc_weights(scores_flat, ti_flat, tf_flat):
    return pl.kernel(
        _sc_weights_body,
        out_type=jax.ShapeDtypeStruct((4 * _T,), jnp.float32),
        mesh=plsc.VectorSubcoreMesh(core_axis_name="c", subcore_axis_name="s"),
        scratch_types=[
            pltpu.VMEM((_CHUNK,), jnp.float32),
            pltpu.VMEM((_CHUNK,), jnp.float32),
            pltpu.VMEM((_L,), jnp.int32),
            pltpu.VMEM((_L,), jnp.float32),
            pltpu.VMEM((8 * _L,), jnp.float32),
            pltpu.VMEM_SHARED((2 * 8 * _L,), jnp.float32),
        ],
    )(scores_flat, ti_flat, tf_flat)


def _scale_kernel(x_ref, w_ref, o_ref):
    o_ref[...] = x_ref[...] * w_ref[...]


def kernel(x, W1, b1, w2, b2):
    B, T, D = x.shape
    del b2
    xf = x.reshape(B * T, D)

    BT = 2048
    scores = pl.pallas_call(
        _scores_kernel,
        grid=(B * T // BT,),
        in_specs=[
            pl.BlockSpec((BT, D), lambda i: (i, 0)),
            pl.BlockSpec((D, D), lambda i: (0, 0)),
            pl.BlockSpec((1, D), lambda i: (0, 0)),
            pl.BlockSpec((1, D), lambda i: (0, 0)),
        ],
        out_specs=pl.BlockSpec((BT, 1), lambda i: (i, 0)),
        out_shape=jax.ShapeDtypeStruct((B * T, 1), jnp.float32),
    )(xf, W1, b1.reshape(1, D), w2.reshape(1, D))

    ti, tf = pl.pallas_call(
        _thresh_kernel,
        out_shape=(jax.ShapeDtypeStruct((B, _L), jnp.int32),
                   jax.ShapeDtypeStruct((B, _L), jnp.float32)),
    )(scores.reshape(B, T // 128, 128))
    weights = _sc_weights(scores.reshape(B * T), ti.reshape(B * _L),
                          tf.reshape(B * _L)).reshape(B, T)

    R = 1024
    out = pl.pallas_call(
        _scale_kernel,
        grid=(B * T // R,),
        in_specs=[
            pl.BlockSpec((R, D), lambda i: (i, 0)),
            pl.BlockSpec((R, 1), lambda i: (i, 0)),
        ],
        out_specs=pl.BlockSpec((R, D), lambda i: (i, 0)),
        out_shape=jax.ShapeDtypeStruct((B * T, D), jnp.float32),
    )(xf, weights.reshape(B * T, 1))

    return out.reshape(B, T, D), weights


# speedup vs baseline: 1.1994x; 1.0240x over previous
import jax
import jax.numpy as jnp
from jax import lax
from jax.experimental import pallas as pl
from jax.experimental.pallas import tpu as pltpu
from jax.experimental.pallas import tpu_sc as plsc

_INT_MIN = -2147483648
_INT_MAX = 2147483647
_L = 16
_T = 8192
_CHUNK = 1024


def _scores_kernel(x_ref, w1_ref, b1_ref, w2_ref, out_ref):
    x = x_ref[...].astype(jnp.bfloat16)
    h = jnp.tanh(
        jax.lax.dot_general(x, w1_ref[...].astype(jnp.bfloat16),
                            (((1,), (1,)), ((), ())),
                            preferred_element_type=jnp.float32)
        + b1_ref[...])
    hb = h.astype(jnp.bfloat16).astype(jnp.float32)
    v2 = w2_ref[...].astype(jnp.bfloat16).astype(jnp.float32)
    s = jnp.sum(hb * v2, axis=1)
    out_ref[...] = s[:, None]


def _thresh_kernel(s_ref, ti_ref, tf_ref):
    s = s_ref[...]
    B = s.shape[0]
    T = s.shape[1] * s.shape[2]
    n_mask = T // 2
    bits = jax.lax.bitcast_convert_type(s, jnp.int32)
    key = jnp.where(bits >= 0, bits, jnp.int32(_INT_MIN) - bits)

    def vbody(_, lohi):
        lo, hi = lohi
        mid = (lo >> 1) + (hi >> 1) + (lo & hi & 1)
        c = jnp.sum((key <= mid).astype(jnp.int32), axis=(1, 2),
                    keepdims=True)
        ge = c >= n_mask
        return (jnp.where(ge, lo, mid + 1), jnp.where(ge, mid, hi))

    lo0 = jnp.full((B, 1, 1), _INT_MIN, jnp.int32)
    hi0 = jnp.full((B, 1, 1), _INT_MAX, jnp.int32)
    K, _ = jax.lax.fori_loop(0, 32, vbody, (lo0, hi0))

    cnt_lt = jnp.sum((key < K).astype(jnp.int32), axis=(1, 2), keepdims=True)
    n_eq = n_mask - cnt_lt
    i1 = jax.lax.broadcasted_iota(jnp.int32, s.shape, 1)
    i2 = jax.lax.broadcasted_iota(jnp.int32, s.shape, 2)
    gidx = i1 * s.shape[2] + i2
    eq = key == K

    def ibody(_, lohi):
        lo, hi = lohi
        mid = (lo + hi) >> 1
        c = jnp.sum((eq & (gidx <= mid)).astype(jnp.int32), axis=(1, 2),
                    keepdims=True)
        ge = c >= n_eq
        return (jnp.where(ge, lo, mid + 1), jnp.where(ge, mid, hi))

    lo0i = jnp.zeros((B, 1, 1), jnp.int32)
    hi0i = jnp.full((B, 1, 1), T - 1, jnp.int32)
    I, _ = jax.lax.fori_loop(0, 13, ibody, (lo0i, hi0i))

    M = jnp.max(s, axis=(1, 2), keepdims=True)
    lane = jax.lax.broadcasted_iota(jnp.int32, (B, _L), 1)
    Kb = jnp.broadcast_to(K[:, :, 0], (B, _L))
    Ib = jnp.broadcast_to(I[:, :, 0], (B, _L))
    Mb = jnp.broadcast_to(M[:, :, 0], (B, _L))
    thr_bits = jnp.where(Kb >= 0, Kb, jnp.int32(_INT_MIN) - Kb)
    thr_f = jax.lax.bitcast_convert_type(thr_bits, jnp.float32)
    ti_ref[...] = jnp.where(lane == 0, Kb, jnp.where(lane == 1, Ib, 0))
    tf_ref[...] = jnp.where(lane == 0, Mb, jnp.where(lane == 1, thr_f, 0.0))


def _sc_weights_body(scores_hbm, ti_hbm, tf_hbm, out_hbm, sv, ev,
                     stg_i, stg_f, stg8f, sp_sum):
    c = lax.axis_index("c")
    s_id = lax.axis_index("s")
    rc = s_id // 8
    j = s_id % 8
    r = 2 * c + rc
    chunk_base = pl.multiple_of(r * _T + j * _CHUNK, _CHUNK)
    n_vec = _CHUNK // _L

    pltpu.sync_copy(scores_hbm.at[pl.ds(chunk_base, _CHUNK)], sv)
    pltpu.sync_copy(ti_hbm.at[pl.ds(pl.multiple_of(r * _L, _L), _L)], stg_i)
    pltpu.sync_copy(tf_hbm.at[pl.ds(pl.multiple_of(r * _L, _L), _L)], stg_f)
    tvec_i = stg_i[pl.ds(0, _L)]
    I = tvec_i[1]
    tvec_f = stg_f[pl.ds(0, _L)]
    M = tvec_f[0]
    thrf = tvec_f[1]

    def _lane_reduce(vec, op):
        acc = vec[0]
        for ln in range(1, _L):
            acc = op(acc, vec[ln])
        return acc

    def body_exp(i, acc):
        v = sv[pl.ds(i * _L, _L)]
        gidx = lax.iota(jnp.int32, _L) + (j * _CHUNK + i * _L)
        masked = (v < thrf) | ((v == thrf) & (gidx <= I))
        e = jnp.where(masked, 0.0, jnp.exp(v - M))
        ev[pl.ds(i * _L, _L)] = e
        return acc + e

    sumv = lax.fori_loop(0, n_vec, body_exp, jnp.zeros((_L,), jnp.float32))
    stg_f[pl.ds(0, _L)] = jnp.full((_L,), _lane_reduce(sumv, jnp.add),
                                   jnp.float32)
    pltpu.sync_copy(stg_f, sp_sum.at[pl.ds(pl.multiple_of((rc * 8 + j) * _L, _L), _L)])
    plsc.subcore_barrier()

    pltpu.sync_copy(sp_sum.at[pl.ds(pl.multiple_of(rc * 8 * _L, _L), 8 * _L)], stg8f)
    S = stg8f[pl.ds(0, _L)][0]
    for jj in range(1, 8):
        S = S + stg8f[pl.ds(jj * _L, _L)][0]
    svec = jnp.full((_L,), S, jnp.float32)

    def body_norm(i, carry):
        ev[pl.ds(i * _L, _L)] = ev[pl.ds(i * _L, _L)] / svec
        return carry

    lax.fori_loop(0, n_vec, body_norm, jnp.int32(0))
    pltpu.sync_copy(ev, out_hbm.at[pl.ds(chunk_base, _CHUNK)])


def _sc_weights(scores_flat, ti_flat, tf_flat):
    return pl.kernel(
        _sc_weights_body,
        out_type=jax.ShapeDtypeStruct((4 * _T,), jnp.float32),
        mesh=plsc.VectorSubcoreMesh(core_axis_name="c", subcore_axis_name="s"),
        scratch_types=[
            pltpu.VMEM((_CHUNK,), jnp.float32),
            pltpu.VMEM((_CHUNK,), jnp.float32),
            pltpu.VMEM((_L,), jnp.int32),
            pltpu.VMEM((_L,), jnp.float32),
            pltpu.VMEM((8 * _L,), jnp.float32),
            pltpu.VMEM_SHARED((2 * 8 * _L,), jnp.float32),
        ],
    )(scores_flat, ti_flat, tf_flat)


def _scale_kernel(x_ref, w_ref, o_ref):
    o_ref[...] = x_ref[...] * w_ref[...]


def kernel(x, W1, b1, w2, b2):
    B, T, D = x.shape
    del b2
    xf = x.reshape(B * T, D)

    BT = 4096
    scores = pl.pallas_call(
        _scores_kernel,
        grid=(B * T // BT,),
        in_specs=[
            pl.BlockSpec((BT, D), lambda i: (i, 0)),
            pl.BlockSpec((D, D), lambda i: (0, 0)),
            pl.BlockSpec((1, D), lambda i: (0, 0)),
            pl.BlockSpec((1, D), lambda i: (0, 0)),
        ],
        out_specs=pl.BlockSpec((BT, 1), lambda i: (i, 0)),
        out_shape=jax.ShapeDtypeStruct((B * T, 1), jnp.float32),
    )(xf, W1, b1.reshape(1, D), w2.reshape(1, D))

    ti, tf = pl.pallas_call(
        _thresh_kernel,
        out_shape=(jax.ShapeDtypeStruct((B, _L), jnp.int32),
                   jax.ShapeDtypeStruct((B, _L), jnp.float32)),
    )(scores.reshape(B, T // 128, 128))
    weights = _sc_weights(scores.reshape(B * T), ti.reshape(B * _L),
                          tf.reshape(B * _L)).reshape(B, T)

    R = 2048
    out = pl.pallas_call(
        _scale_kernel,
        grid=(B * T // R,),
        in_specs=[
            pl.BlockSpec((R, D), lambda i: (i, 0)),
            pl.BlockSpec((R, 1), lambda i: (i, 0)),
        ],
        out_specs=pl.BlockSpec((R, D), lambda i: (i, 0)),
        out_shape=jax.ShapeDtypeStruct((B * T, D), jnp.float32),
    )(xf, weights.reshape(B * T, 1))

    return out.reshape(B, T, D), weights
